# baseline (speedup 1.0000x reference)
import functools

import jax
import jax.numpy as jnp
from jax import lax
from jax.experimental import pallas as pl
from jax.experimental.pallas import tpu as pltpu
from jax.experimental.pallas import tpu_sc as plsc

NC = 2
NS = 16
NW = NC * NS
L = 16

ROWS = 160
BCH = ROWS * 128
UNROLL = 8


def _make_kernel(M, B_pad):
    base_sz = (M // NW) // 8 * 8
    last_sz = M - (NW - 1) * base_sz
    assert last_sz % 8 == 0 and last_sz >= base_sz
    n_chunks = B_pad // BCH
    assert B_pad % BCH == 0 and L * UNROLL == 128

    mesh = plsc.VectorSubcoreMesh(
        core_axis_name="c", subcore_axis_name="s", num_cores=NC, num_subcores=NS
    )

    @functools.partial(
        pl.kernel,
        out_type=jax.ShapeDtypeStruct((M,), jnp.int32),
        mesh=mesh,
        scratch_types=[
            pltpu.VMEM((last_sz + 8,), jnp.int32),
            pltpu.VMEM((BCH,), jnp.int32),
            pltpu.VMEM((BCH,), jnp.int32),
            pltpu.VMEM((ROWS, 128), jnp.int32),
            pltpu.VMEM((ROWS, 128), jnp.int32),
            pltpu.VMEM_SHARED((ROWS, 128), jnp.int32),
            pltpu.VMEM_SHARED((ROWS, 128), jnp.int32),
            pltpu.SemaphoreType.DMA,
            pltpu.SemaphoreType.DMA,
            pltpu.SemaphoreType.DMA,
            pltpu.SemaphoreType.DMA,
        ],
        compiler_params=pltpu.CompilerParams(needs_layout_passes=False),
    )
    def scatter_kernel(in_hbm, idx_hbm, val_hbm, out_hbm,
                       local, idxb0, idxb1, valb0, valb1, sval0, sval1,
                       isem0, isem1, vsem0, vsem1):
        idxbufs = [idxb0, idxb1]
        valbufs = [valb0, valb1]
        svals = [sval0, sval1]
        isems = [isem0, isem1]
        vsems = [vsem0, vsem1]
        sid = lax.axis_index("s")
        wid = sid * NC + lax.axis_index("c")
        base = wid * base_sz
        is_last = wid == NW - 1
        n_local = jnp.where(is_last, last_sz, base_sz)
        vbase = jnp.full((L,), base, jnp.int32)
        vn = jnp.full((L,), n_local, jnp.uint32)

        def start_idx_fetch(c):
            slot = c % 2
            pltpu.async_copy(idx_hbm.at[pl.ds(c * BCH, BCH)], idxbufs[slot],
                             isems[slot])

        def wait_idx_fetch(c):
            slot = c % 2
            pltpu.make_async_copy(idx_hbm.at[pl.ds(c * BCH, BCH)],
                                  idxbufs[slot], isems[slot]).wait()

        def stage_val(c):
            slot = c % 2
            pltpu.sync_copy(val_hbm.at[pl.ds(c * ROWS, ROWS), :], svals[slot])

        @pl.when(sid == 0)
        def _():
            stage_val(0)

        start_idx_fetch(0)

        @pl.when(jnp.logical_not(is_last))
        def _():
            pltpu.sync_copy(in_hbm.at[pl.ds(base, base_sz)],
                            local.at[pl.ds(0, base_sz)])

        @pl.when(is_last)
        def _():
            pltpu.sync_copy(in_hbm.at[pl.ds(base, last_sz)],
                            local.at[pl.ds(0, last_sz)])

        if n_chunks > 1:
            @pl.when(sid == 0)
            def _():
                stage_val(1)

        for c in range(n_chunks):
            slot = c % 2

            plsc.subcore_barrier()
            pltpu.async_copy(svals[slot], valbufs[slot], vsems[slot])
            wait_idx_fetch(c)
            if c + 1 < n_chunks:
                start_idx_fetch(c + 1)
            pltpu.make_async_copy(svals[slot], valbufs[slot],
                                  vsems[slot]).wait()
            plsc.subcore_barrier()
            if c + 2 < n_chunks:
                @pl.when(sid == 0)
                def _():
                    stage_val(c + 2)

            idxb = idxbufs[slot]
            valb = valbufs[slot]

            def body(j, carry):
                locs, vals = [], []
                for u in range(UNROLL):
                    off = pl.multiple_of(j * 128 + u * L, L)
                    loc = plsc.bitcast(idxb[pl.ds(off, L)] - vbase, jnp.uint32)
                    locs.append(plsc.bitcast(jnp.minimum(loc, vn), jnp.int32))
                    vals.append(valb[j, pl.ds(u * L, L)])
                for u in range(UNROLL):
                    plsc.store_scatter(local, [locs[u]], vals[u])
                return carry

            lax.fori_loop(0, ROWS, body, 0)

        @pl.when(jnp.logical_not(is_last))
        def _():
            pltpu.sync_copy(local.at[pl.ds(0, base_sz)],
                            out_hbm.at[pl.ds(base, base_sz)])

        @pl.when(is_last)
        def _():
            pltpu.sync_copy(local.at[pl.ds(0, last_sz)],
                            out_hbm.at[pl.ds(base, last_sz)])

    return scatter_kernel


@jax.jit
def kernel(input, index, value):
    M = input.shape[0]
    B = index.shape[0]
    B_pad = -(-B // BCH) * BCH
    idx_pad = jnp.concatenate(
        [index.astype(jnp.int32), jnp.full((B_pad - B,), M, jnp.int32)])
    val_pad = jnp.concatenate(
        [value.astype(jnp.int32), jnp.zeros((B_pad - B,), jnp.int32)])
    out = _make_kernel(M, B_pad)(
        input.astype(jnp.int32), idx_pad, val_pad.reshape(B_pad // 128, 128)
    )
    return out.astype(input.dtype)


# speedup vs baseline: 1.1329x; 1.1329x over previous
import functools

import jax
import jax.numpy as jnp
from jax import lax
from jax.experimental import pallas as pl
from jax.experimental.pallas import tpu as pltpu
from jax.experimental.pallas import tpu_sc as plsc

NC = 2
NS = 16
NW = NC * NS
L = 16

ROWS = 160
BCH = ROWS * 128
UNROLL = 8


def _make_kernel(M, B_pad):
    base_sz = (M // NW) // 8 * 8
    last_sz = M - (NW - 1) * base_sz
    assert last_sz % 8 == 0 and last_sz >= base_sz
    n_chunks = B_pad // BCH
    assert B_pad % BCH == 0 and L * UNROLL == 128

    mesh = plsc.VectorSubcoreMesh(
        core_axis_name="c", subcore_axis_name="s", num_cores=NC, num_subcores=NS
    )

    @functools.partial(
        pl.kernel,
        out_type=jax.ShapeDtypeStruct((M,), jnp.int32),
        mesh=mesh,
        scratch_types=[
            pltpu.VMEM((last_sz + 8,), jnp.int32),
            pltpu.VMEM((BCH,), jnp.int32),
            pltpu.VMEM((BCH,), jnp.int32),
            pltpu.VMEM((ROWS, 128), jnp.int32),
            pltpu.VMEM((ROWS, 128), jnp.int32),
            pltpu.VMEM_SHARED((ROWS, 128), jnp.int32),
            pltpu.VMEM_SHARED((ROWS, 128), jnp.int32),
            pltpu.SemaphoreType.DMA,
            pltpu.SemaphoreType.DMA,
            pltpu.SemaphoreType.DMA,
            pltpu.SemaphoreType.DMA,
            pltpu.SemaphoreType.DMA,
            pltpu.SemaphoreType.DMA,
        ],
        compiler_params=pltpu.CompilerParams(needs_layout_passes=False),
    )
    def scatter_kernel(in_hbm, idx_hbm, val_hbm, out_hbm,
                       local, idxb0, idxb1, valb0, valb1, sval0, sval1,
                       isem0, isem1, vsem0, vsem1, ssem0, ssem1):
        idxbufs = [idxb0, idxb1]
        valbufs = [valb0, valb1]
        svals = [sval0, sval1]
        isems = [isem0, isem1]
        vsems = [vsem0, vsem1]
        ssems = [ssem0, ssem1]
        sid = lax.axis_index("s")
        wid = sid * NC + lax.axis_index("c")
        base = wid * base_sz
        is_last = wid == NW - 1
        n_local = jnp.where(is_last, last_sz, base_sz)
        vbase = jnp.full((L,), base, jnp.int32)
        vn = jnp.full((L,), n_local, jnp.uint32)

        def start_idx_fetch(c):
            slot = c % 2
            pltpu.async_copy(idx_hbm.at[pl.ds(c * BCH, BCH)], idxbufs[slot],
                             isems[slot])

        def wait_idx_fetch(c):
            slot = c % 2
            pltpu.make_async_copy(idx_hbm.at[pl.ds(c * BCH, BCH)],
                                  idxbufs[slot], isems[slot]).wait()

        def start_stage_val(c):
            slot = c % 2
            pltpu.async_copy(val_hbm.at[pl.ds(c * ROWS, ROWS), :],
                             svals[slot], ssems[slot])

        def wait_stage_val(c):
            slot = c % 2
            pltpu.make_async_copy(val_hbm.at[pl.ds(c * ROWS, ROWS), :],
                                  svals[slot], ssems[slot]).wait()

        def start_pull_val(c):
            slot = c % 2
            pltpu.async_copy(svals[slot], valbufs[slot], vsems[slot])

        def wait_pull_val(c):
            slot = c % 2
            pltpu.make_async_copy(svals[slot], valbufs[slot],
                                  vsems[slot]).wait()

        @pl.when(sid == 0)
        def _():
            start_stage_val(0)

        start_idx_fetch(0)

        @pl.when(jnp.logical_not(is_last))
        def _():
            pltpu.sync_copy(in_hbm.at[pl.ds(base, base_sz)],
                            local.at[pl.ds(0, base_sz)])

        @pl.when(is_last)
        def _():
            pltpu.sync_copy(in_hbm.at[pl.ds(base, last_sz)],
                            local.at[pl.ds(0, last_sz)])

        @pl.when(sid == 0)
        def _():
            wait_stage_val(0)

        plsc.subcore_barrier()
        if n_chunks > 1:
            @pl.when(sid == 0)
            def _():
                start_stage_val(1)
        start_pull_val(0)

        for c in range(n_chunks):
            slot = c % 2
            wait_pull_val(c)
            if c + 1 < n_chunks:
                @pl.when(sid == 0)
                def _():
                    wait_stage_val(c + 1)
            plsc.subcore_barrier()
            if c + 2 < n_chunks:
                @pl.when(sid == 0)
                def _():
                    start_stage_val(c + 2)
            if c + 1 < n_chunks:
                start_pull_val(c + 1)
            wait_idx_fetch(c)
            if c + 1 < n_chunks:
                start_idx_fetch(c + 1)

            idxb = idxbufs[slot]
            valb = valbufs[slot]

            def body(j, carry):
                locs, vals = [], []
                for u in range(UNROLL):
                    off = pl.multiple_of(j * 128 + u * L, L)
                    loc = plsc.bitcast(idxb[pl.ds(off, L)] - vbase, jnp.uint32)
                    locs.append(plsc.bitcast(jnp.minimum(loc, vn), jnp.int32))
                    vals.append(valb[j, pl.ds(u * L, L)])
                for u in range(UNROLL):
                    plsc.store_scatter(local, [locs[u]], vals[u])
                return carry

            lax.fori_loop(0, ROWS, body, 0)

        @pl.when(jnp.logical_not(is_last))
        def _():
            pltpu.sync_copy(local.at[pl.ds(0, base_sz)],
                            out_hbm.at[pl.ds(base, base_sz)])

        @pl.when(is_last)
        def _():
            pltpu.sync_copy(local.at[pl.ds(0, last_sz)],
                            out_hbm.at[pl.ds(base, last_sz)])

    return scatter_kernel


@jax.jit
def kernel(input, index, value):
    M = input.shape[0]
    B = index.shape[0]
    B_pad = -(-B // BCH) * BCH
    idx_pad = jnp.concatenate(
        [index.astype(jnp.int32), jnp.full((B_pad - B,), M, jnp.int32)])
    val_pad = jnp.concatenate(
        [value.astype(jnp.int32), jnp.zeros((B_pad - B,), jnp.int32)])
    out = _make_kernel(M, B_pad)(
        input.astype(jnp.int32), idx_pad, val_pad.reshape(B_pad // 128, 128)
    )
    return out.astype(input.dtype)
